# baseline (speedup 1.0000x reference)
import numpy as np

import jax
import jax.numpy as jnp
from jax.experimental import pallas as pl
from jax.experimental.pallas import tpu as pltpu

_H, _W = 128, 256
_L, _C = 5, 64
_K = (_H * _W) // 2
_CB = 16


def _gauss_weights(k_size=5, sigma=1.0):
    center = k_size // 2
    gx, gy = np.mgrid[0 - center:k_size - center, 0 - center:k_size - center]
    g = 1.0 / (2.0 * np.pi * sigma) * np.exp(-(np.square(gx) + np.square(gy)) / (2.0 * np.square(sigma)))
    return g.astype(np.float32)


_GWB = _gauss_weights().astype(jnp.bfloat16).astype(np.float32)


def _search_bits(keys, prefix, bits):
    for bit in bits:
        cand = prefix | (1 << bit)
        cnt = jnp.sum((keys >= cand).astype(jnp.int32))
        prefix = jnp.where(cnt >= _K, cand, prefix)
    return prefix


def _fused(conf_ref, tgt_ref, x_ref, oc_ref, og_ref, rate_ref,
           mc_s, mg_s, keys_s, st_s):
    H, W, K = _H, _W, _K
    l = pl.program_id(0)
    c = pl.program_id(1)

    @pl.when((l == 0) & (c == 0))
    def _():
        ones = jnp.ones((H, W), jnp.float32)
        mc_s[...] = ones
        mg_s[...] = ones

    xv = x_ref[...]
    oc_ref[...] = xv * mc_s[...][None, None]
    og_ref[...] = xv * mg_s[...][None, None]

    @pl.when((l < 4) & (c == 0))
    def _():
        cc = conf_ref[0]
        s = jnp.maximum(jax.nn.sigmoid(cc[0]), jax.nn.sigmoid(cc[1]))
        sb = s.astype(jnp.bfloat16).astype(jnp.float32)
        zrow = jnp.zeros((2, W), jnp.float32)
        zcol = jnp.zeros((H + 4, 2), jnp.float32)
        sp = jnp.concatenate([zrow, sb, zrow], axis=0)
        sp = jnp.concatenate([zcol, sp, zcol], axis=1)
        acc = jnp.zeros((H, W), jnp.float32)
        for dy in range(5):
            for dx in range(5):
                acc = acc + _GWB[dy, dx] * jax.lax.slice(
                    sp, (dy, dx), (dy + H, dx + W))
        keys = jax.lax.bitcast_convert_type(acc, jnp.int32)
        keys_s[...] = keys
        st_s[0] = _search_bits(keys, jnp.int32(0), range(29, 21, -1))

    @pl.when((l < 4) & (c == 1))
    def _():
        st_s[0] = _search_bits(keys_s[...], st_s[0], range(21, 10, -1))

    @pl.when((l < 4) & (c == 2))
    def _():
        st_s[0] = _search_bits(keys_s[...], st_s[0], range(10, -1, -1))

    @pl.when((l < 4) & (c == 3))
    def _():
        keys = keys_s[...]
        thresh = st_s[0]
        gcnt = jnp.sum((keys > thresh).astype(jnp.int32))
        need = K - gcnt
        tie = keys == thresh
        fidx = (jax.lax.broadcasted_iota(jnp.int32, (H, W), 0) * W
                + jax.lax.broadcasted_iota(jnp.int32, (H, W), 1))
        P = jnp.int32(0)
        for bit in range(14, -1, -1):
            mid = P | (1 << bit)
            cnt = jnp.sum((tie & (fidx < mid)).astype(jnp.int32))
            P = jnp.where(cnt >= need, P, mid)
        mc_s[...] = ((keys > thresh) | (tie & (fidx <= P))).astype(jnp.float32)

    @pl.when((l == 0) & (c == 3))
    def _():
        ys = jax.lax.broadcasted_iota(jnp.int32, (H, W), 0)
        xs = jax.lax.broadcasted_iota(jnp.int32, (H, W), 1)
        gt = jnp.zeros((H, W), jnp.bool_)
        for i in range(10):
            x1 = jnp.maximum(tgt_ref[i, 0], 0)
            y1 = jnp.maximum(tgt_ref[i, 1], 0)
            x2 = jnp.minimum(tgt_ref[i, 2], W)
            y2 = jnp.minimum(tgt_ref[i, 3], H)
            gt = gt | ((ys >= y1) & (ys < y2) & (xs >= x1) & (xs < x2))
        gtf = gt.astype(jnp.float32)
        rate_ref[0, 0] = 0.5 + jnp.sum(gtf) / float(H * W)
        mg_s[...] = gtf


def kernel(x, confidence_maps, targets_label, B):
    H, W, L, C, Cb = _H, _W, _L, _C, _CB
    xc, xg, rate = pl.pallas_call(
        _fused,
        grid=(L, C // Cb),
        in_specs=[
            pl.BlockSpec((1, 2, H, W),
                         lambda l, c: (jnp.minimum(l + 1, L - 1), 0, 0, 0)),
            pl.BlockSpec(memory_space=pltpu.SMEM),
            pl.BlockSpec((1, Cb, H, W), lambda l, c: (l, c, 0, 0)),
        ],
        out_specs=(
            pl.BlockSpec((1, Cb, H, W), lambda l, c: (l, c, 0, 0)),
            pl.BlockSpec((1, Cb, H, W), lambda l, c: (l, c, 0, 0)),
            pl.BlockSpec(memory_space=pltpu.SMEM),
        ),
        out_shape=(
            jax.ShapeDtypeStruct((L, C, H, W), jnp.float32),
            jax.ShapeDtypeStruct((L, C, H, W), jnp.float32),
            jax.ShapeDtypeStruct((1, 1), jnp.float32),
        ),
        scratch_shapes=[
            pltpu.VMEM((H, W), jnp.float32),
            pltpu.VMEM((H, W), jnp.float32),
            pltpu.VMEM((H, W), jnp.int32),
            pltpu.SMEM((4,), jnp.int32),
        ],
    )(confidence_maps, targets_label, x)
    return xc, xg, rate[0, 0]


# speedup vs baseline: 1.1435x; 1.1435x over previous
import numpy as np

import jax
import jax.numpy as jnp
from jax.experimental import pallas as pl
from jax.experimental.pallas import tpu as pltpu

_H, _W = 128, 256
_L, _C = 5, 64
_K = (_H * _W) // 2


def _gauss_weights(k_size=5, sigma=1.0):
    center = k_size // 2
    gx, gy = np.mgrid[0 - center:k_size - center, 0 - center:k_size - center]
    g = 1.0 / (2.0 * np.pi * sigma) * np.exp(-(np.square(gx) + np.square(gy)) / (2.0 * np.square(sigma)))
    return g.astype(np.float32)


_GW = _gauss_weights()
_GWB = _GW.astype(jnp.bfloat16).astype(np.float32)


def _band_matrices():
    B = np.zeros((5, _W, _W), np.float32)
    for dy in range(5):
        for d in range(-2, 3):
            for x in range(_W):
                k = x + d
                if 0 <= k < _W:
                    B[dy, k, x] = _GWB[dy, d + 2]
    return B


_BNP = _band_matrices()


def _mask_stage(conf_ref, tgt_ref, b_ref, mconf_ref, mgt_ref, rate_ref):
    H, W, K = _H, _W, _K
    c = conf_ref[...]
    s = jnp.maximum(jax.nn.sigmoid(c[:, 0]), jax.nn.sigmoid(c[:, 1]))
    sb = s[1:5].astype(jnp.bfloat16).astype(jnp.float32)
    zrow = jnp.zeros((4, 2, W), jnp.float32)
    sp = jnp.concatenate([zrow, sb, zrow], axis=1)
    acc = jnp.zeros((4, H, W), jnp.float32)
    for dy in range(5):
        sv = jax.lax.slice(sp, (0, dy, 0), (4, dy + H, W)).astype(jnp.bfloat16)
        acc = acc + jax.lax.dot_general(
            sv, b_ref[dy], (((2,), (0,)), ((), ())),
            preferred_element_type=jnp.float32)
    keys = jax.lax.bitcast_convert_type(acc, jnp.int32)
    prefix = jnp.zeros((4, 1, 1), jnp.int32)
    for bit in range(29, -1, -1):
        cand = prefix | (1 << bit)
        cnt = jnp.sum((keys >= cand).astype(jnp.int32), axis=(1, 2), keepdims=True)
        prefix = jnp.where(cnt >= K, cand, prefix)
    thresh = prefix
    gcnt = jnp.sum((keys > thresh).astype(jnp.int32), axis=(1, 2), keepdims=True)
    need = K - gcnt
    tie = keys == thresh
    fidx = (jax.lax.broadcasted_iota(jnp.int32, (H, W), 0) * W
            + jax.lax.broadcasted_iota(jnp.int32, (H, W), 1))[None]
    P = jnp.zeros((4, 1, 1), jnp.int32)
    for bit in range(14, -1, -1):
        mid = P | (1 << bit)
        cnt = jnp.sum((tie & (fidx < mid)).astype(jnp.int32), axis=(1, 2), keepdims=True)
        P = jnp.where(cnt >= need, P, mid)
    mask = (keys > thresh) | (tie & (fidx <= P))
    mconf_ref[0, 0] = jnp.ones((H, W), jnp.float32)
    mconf_ref[1:5, 0] = mask.astype(jnp.float32)

    ys = jax.lax.broadcasted_iota(jnp.int32, (H, W), 0)
    xs = jax.lax.broadcasted_iota(jnp.int32, (H, W), 1)
    gt = jnp.zeros((H, W), jnp.bool_)
    for i in range(10):
        x1 = jnp.maximum(tgt_ref[i, 0], 0)
        y1 = jnp.maximum(tgt_ref[i, 1], 0)
        x2 = jnp.minimum(tgt_ref[i, 2], W)
        y2 = jnp.minimum(tgt_ref[i, 3], H)
        gt = gt | ((ys >= y1) & (ys < y2) & (xs >= x1) & (xs < x2))
    gtf = gt.astype(jnp.float32)
    mgt_ref[0, 0] = jnp.ones((H, W), jnp.float32)
    mgt_ref[1:5, 0] = jnp.broadcast_to(gtf[None], (4, H, W))
    rate_ref[0, 0] = 0.5 + jnp.sum(gtf) / float(H * W)


def _apply_stage(x_ref, mc_ref, mg_ref, oc_ref, og_ref):
    xv = x_ref[...]
    oc_ref[...] = xv * mc_ref[...]
    og_ref[...] = xv * mg_ref[...]


def kernel(x, confidence_maps, targets_label, B):
    H, W, L, C = _H, _W, _L, _C
    mconf, mgt, rate = pl.pallas_call(
        _mask_stage,
        out_shape=(
            jax.ShapeDtypeStruct((L, 1, H, W), jnp.float32),
            jax.ShapeDtypeStruct((L, 1, H, W), jnp.float32),
            jax.ShapeDtypeStruct((1, 1), jnp.float32),
        ),
        in_specs=[
            pl.BlockSpec(memory_space=pltpu.VMEM),
            pl.BlockSpec(memory_space=pltpu.SMEM),
            pl.BlockSpec(memory_space=pltpu.VMEM),
        ],
        out_specs=(
            pl.BlockSpec(memory_space=pltpu.VMEM),
            pl.BlockSpec(memory_space=pltpu.VMEM),
            pl.BlockSpec(memory_space=pltpu.SMEM),
        ),
    )(confidence_maps, targets_label, jnp.asarray(_BNP, jnp.bfloat16))

    Cb = 64
    xc, xg = pl.pallas_call(
        _apply_stage,
        grid=(L, C // Cb),
        compiler_params=pltpu.CompilerParams(
            dimension_semantics=("parallel", "parallel")),
        in_specs=[
            pl.BlockSpec((1, Cb, H, W), lambda l, c: (l, c, 0, 0)),
            pl.BlockSpec((1, 1, H, W), lambda l, c: (l, 0, 0, 0)),
            pl.BlockSpec((1, 1, H, W), lambda l, c: (l, 0, 0, 0)),
        ],
        out_specs=(
            pl.BlockSpec((1, Cb, H, W), lambda l, c: (l, c, 0, 0)),
            pl.BlockSpec((1, Cb, H, W), lambda l, c: (l, c, 0, 0)),
        ),
        out_shape=(
            jax.ShapeDtypeStruct((L, C, H, W), jnp.float32),
            jax.ShapeDtypeStruct((L, C, H, W), jnp.float32),
        ),
    )(x, mconf, mgt)
    return xc, xg, rate[0, 0]


# speedup vs baseline: 1.1723x; 1.0252x over previous
import numpy as np

import jax
import jax.numpy as jnp
from jax.experimental import pallas as pl
from jax.experimental.pallas import tpu as pltpu

_H, _W = 128, 256
_L, _C = 5, 64
_K = (_H * _W) // 2


def _gauss_weights(k_size=5, sigma=1.0):
    center = k_size // 2
    gx, gy = np.mgrid[0 - center:k_size - center, 0 - center:k_size - center]
    g = 1.0 / (2.0 * np.pi * sigma) * np.exp(-(np.square(gx) + np.square(gy)) / (2.0 * np.square(sigma)))
    return g.astype(np.float32)


_GW = _gauss_weights()
_GWB = _GW.astype(jnp.bfloat16).astype(np.float32)


def _band_matrices():
    B = np.zeros((5, _W, _W), np.float32)
    for dy in range(5):
        for d in range(-2, 3):
            for x in range(_W):
                k = x + d
                if 0 <= k < _W:
                    B[dy, k, x] = _GWB[dy, d + 2]
    return B


_BNP = _band_matrices()


def _mask_stage(conf_ref, tgt_ref, b_ref, mconf_ref, mgt_ref, rate_ref):
    H, W, K = _H, _W, _K
    c = conf_ref[...]
    s = jnp.maximum(jax.nn.sigmoid(c[:, 0]), jax.nn.sigmoid(c[:, 1]))
    sb = s[1:5].astype(jnp.bfloat16).astype(jnp.float32)
    zrow = jnp.zeros((4, 2, W), jnp.float32)
    sp = jnp.concatenate([zrow, sb, zrow], axis=1)
    acc = jnp.zeros((4, H, W), jnp.float32)
    for dy in range(5):
        sv = jax.lax.slice(sp, (0, dy, 0), (4, dy + H, W)).astype(jnp.bfloat16)
        acc = acc + jax.lax.dot_general(
            sv, b_ref[dy], (((2,), (0,)), ((), ())),
            preferred_element_type=jnp.float32)
    keys = jax.lax.bitcast_convert_type(acc, jnp.int32)
    prefix = jnp.zeros((4, 1, 1), jnp.int32)
    for hi in range(29, -1, -2):
        lo = hi - 1
        c10 = prefix | (1 << hi)
        c01 = prefix | (1 << lo)
        c11 = c10 | (1 << lo)
        n10 = jnp.sum((keys >= c10).astype(jnp.int32), axis=(1, 2), keepdims=True)
        n01 = jnp.sum((keys >= c01).astype(jnp.int32), axis=(1, 2), keepdims=True)
        n11 = jnp.sum((keys >= c11).astype(jnp.int32), axis=(1, 2), keepdims=True)
        prefix = jnp.where(n10 >= K,
                           jnp.where(n11 >= K, c11, c10),
                           jnp.where(n01 >= K, c01, prefix))
    thresh = prefix
    gcnt = jnp.sum((keys > thresh).astype(jnp.int32), axis=(1, 2), keepdims=True)
    need = K - gcnt
    tie = keys == thresh
    fidx = (jax.lax.broadcasted_iota(jnp.int32, (H, W), 0) * W
            + jax.lax.broadcasted_iota(jnp.int32, (H, W), 1))[None]
    P = jnp.zeros((4, 1, 1), jnp.int32)
    for bit in range(14, -1, -1):
        mid = P | (1 << bit)
        cnt = jnp.sum((tie & (fidx < mid)).astype(jnp.int32), axis=(1, 2), keepdims=True)
        P = jnp.where(cnt >= need, P, mid)
    mask = (keys > thresh) | (tie & (fidx <= P))
    mconf_ref[0, 0] = jnp.ones((H, W), jnp.float32)
    mconf_ref[1:5, 0] = mask.astype(jnp.float32)

    ys = jax.lax.broadcasted_iota(jnp.int32, (H, W), 0)
    xs = jax.lax.broadcasted_iota(jnp.int32, (H, W), 1)
    gt = jnp.zeros((H, W), jnp.bool_)
    for i in range(10):
        x1 = jnp.maximum(tgt_ref[i, 0], 0)
        y1 = jnp.maximum(tgt_ref[i, 1], 0)
        x2 = jnp.minimum(tgt_ref[i, 2], W)
        y2 = jnp.minimum(tgt_ref[i, 3], H)
        gt = gt | ((ys >= y1) & (ys < y2) & (xs >= x1) & (xs < x2))
    gtf = gt.astype(jnp.float32)
    mgt_ref[0, 0] = jnp.ones((H, W), jnp.float32)
    mgt_ref[1:5, 0] = jnp.broadcast_to(gtf[None], (4, H, W))
    rate_ref[0, 0] = 0.5 + jnp.sum(gtf) / float(H * W)


def _apply_stage(x_ref, mc_ref, mg_ref, oc_ref, og_ref):
    xv = x_ref[...]
    oc_ref[...] = xv * mc_ref[...]
    og_ref[...] = xv * mg_ref[...]


def kernel(x, confidence_maps, targets_label, B):
    H, W, L, C = _H, _W, _L, _C
    mconf, mgt, rate = pl.pallas_call(
        _mask_stage,
        out_shape=(
            jax.ShapeDtypeStruct((L, 1, H, W), jnp.float32),
            jax.ShapeDtypeStruct((L, 1, H, W), jnp.float32),
            jax.ShapeDtypeStruct((1, 1), jnp.float32),
        ),
        in_specs=[
            pl.BlockSpec(memory_space=pltpu.VMEM),
            pl.BlockSpec(memory_space=pltpu.SMEM),
            pl.BlockSpec(memory_space=pltpu.VMEM),
        ],
        out_specs=(
            pl.BlockSpec(memory_space=pltpu.VMEM),
            pl.BlockSpec(memory_space=pltpu.VMEM),
            pl.BlockSpec(memory_space=pltpu.SMEM),
        ),
    )(confidence_maps, targets_label, jnp.asarray(_BNP, jnp.bfloat16))

    Cb = 64
    xc, xg = pl.pallas_call(
        _apply_stage,
        grid=(L, C // Cb),
        compiler_params=pltpu.CompilerParams(
            dimension_semantics=("parallel", "parallel")),
        in_specs=[
            pl.BlockSpec((1, Cb, H, W), lambda l, c: (l, c, 0, 0)),
            pl.BlockSpec((1, 1, H, W), lambda l, c: (l, 0, 0, 0)),
            pl.BlockSpec((1, 1, H, W), lambda l, c: (l, 0, 0, 0)),
        ],
        out_specs=(
            pl.BlockSpec((1, Cb, H, W), lambda l, c: (l, c, 0, 0)),
            pl.BlockSpec((1, Cb, H, W), lambda l, c: (l, c, 0, 0)),
        ),
        out_shape=(
            jax.ShapeDtypeStruct((L, C, H, W), jnp.float32),
            jax.ShapeDtypeStruct((L, C, H, W), jnp.float32),
        ),
    )(x, mconf, mgt)
    return xc, xg, rate[0, 0]


# speedup vs baseline: 1.1817x; 1.0080x over previous
import numpy as np

import jax
import jax.numpy as jnp
from jax.experimental import pallas as pl
from jax.experimental.pallas import tpu as pltpu

_H, _W = 128, 256
_L, _C = 5, 64
_K = (_H * _W) // 2


def _gauss_weights(k_size=5, sigma=1.0):
    center = k_size // 2
    gx, gy = np.mgrid[0 - center:k_size - center, 0 - center:k_size - center]
    g = 1.0 / (2.0 * np.pi * sigma) * np.exp(-(np.square(gx) + np.square(gy)) / (2.0 * np.square(sigma)))
    return g.astype(np.float32)


_GW = _gauss_weights()
_GWB = _GW.astype(jnp.bfloat16).astype(np.float32)


def _band_matrices():
    B = np.zeros((5, _W, _W), np.float32)
    for dy in range(5):
        for d in range(-2, 3):
            for x in range(_W):
                k = x + d
                if 0 <= k < _W:
                    B[dy, k, x] = _GWB[dy, d + 2]
    return B


_BNP = _band_matrices()


def _compute_masks(conf_ref, tgt_ref, b_ref, rate_ref, mcs, mgs):
    H, W, K = _H, _W, _K
    c = conf_ref[...]
    s = jnp.maximum(jax.nn.sigmoid(c[:, 0]), jax.nn.sigmoid(c[:, 1]))
    sb = s[1:5].astype(jnp.bfloat16).astype(jnp.float32)
    zrow = jnp.zeros((4, 2, W), jnp.float32)
    sp = jnp.concatenate([zrow, sb, zrow], axis=1)
    acc = jnp.zeros((4, H, W), jnp.float32)
    for dy in range(5):
        sv = jax.lax.slice(sp, (0, dy, 0), (4, dy + H, W)).astype(jnp.bfloat16)
        acc = acc + jax.lax.dot_general(
            sv, b_ref[dy], (((2,), (0,)), ((), ())),
            preferred_element_type=jnp.float32)
    keys = jax.lax.bitcast_convert_type(acc, jnp.int32)
    prefix = jnp.zeros((4, 1, 1), jnp.int32)
    for hi in range(29, -1, -2):
        lo = hi - 1
        c10 = prefix | (1 << hi)
        c01 = prefix | (1 << lo)
        c11 = c10 | (1 << lo)
        n10 = jnp.sum((keys >= c10).astype(jnp.int32), axis=(1, 2), keepdims=True)
        n01 = jnp.sum((keys >= c01).astype(jnp.int32), axis=(1, 2), keepdims=True)
        n11 = jnp.sum((keys >= c11).astype(jnp.int32), axis=(1, 2), keepdims=True)
        prefix = jnp.where(n10 >= K,
                           jnp.where(n11 >= K, c11, c10),
                           jnp.where(n01 >= K, c01, prefix))
    thresh = prefix
    gcnt = jnp.sum((keys > thresh).astype(jnp.int32), axis=(1, 2), keepdims=True)
    need = K - gcnt
    tie = keys == thresh
    fidx = (jax.lax.broadcasted_iota(jnp.int32, (H, W), 0) * W
            + jax.lax.broadcasted_iota(jnp.int32, (H, W), 1))[None]
    P = jnp.zeros((4, 1, 1), jnp.int32)
    for bit in range(14, -1, -1):
        mid = P | (1 << bit)
        cnt = jnp.sum((tie & (fidx < mid)).astype(jnp.int32), axis=(1, 2), keepdims=True)
        P = jnp.where(cnt >= need, P, mid)
    mcs[...] = ((keys > thresh) | (tie & (fidx <= P))).astype(jnp.float32)

    ys = jax.lax.broadcasted_iota(jnp.int32, (H, W), 0)
    xs = jax.lax.broadcasted_iota(jnp.int32, (H, W), 1)
    gt = jnp.zeros((H, W), jnp.bool_)
    for i in range(10):
        x1 = jnp.maximum(tgt_ref[i, 0], 0)
        y1 = jnp.maximum(tgt_ref[i, 1], 0)
        x2 = jnp.minimum(tgt_ref[i, 2], W)
        y2 = jnp.minimum(tgt_ref[i, 3], H)
        gt = gt | ((ys >= y1) & (ys < y2) & (xs >= x1) & (xs < x2))
    gtf = gt.astype(jnp.float32)
    mgs[...] = gtf
    rate_ref[0, 0] = 0.5 + jnp.sum(gtf) / float(H * W)


_CB = 32


def _fused(conf_ref, tgt_ref, b_ref, x_ref, oc_ref, og_ref, rate_ref, mcs, mgs):
    l = pl.program_id(0)
    c = pl.program_id(1)
    xv = x_ref[...]

    @pl.when(l == 0)
    def _():
        oc_ref[...] = xv
        og_ref[...] = xv

    @pl.when((l == 0) & (c == 0))
    def _():
        _compute_masks(conf_ref, tgt_ref, b_ref, rate_ref, mcs, mgs)

    @pl.when(l > 0)
    def _():
        m = mcs[pl.ds(l - 1, 1), :, :]
        oc_ref[...] = xv * m[None]
        og_ref[...] = xv * mgs[...][None, None]


def kernel(x, confidence_maps, targets_label, B):
    H, W, L, C, Cb = _H, _W, _L, _C, _CB
    xc, xg, rate = pl.pallas_call(
        _fused,
        grid=(L, C // Cb),
        in_specs=[
            pl.BlockSpec(memory_space=pltpu.VMEM),
            pl.BlockSpec(memory_space=pltpu.SMEM),
            pl.BlockSpec(memory_space=pltpu.VMEM),
            pl.BlockSpec((1, Cb, H, W), lambda l, c: (l, c, 0, 0)),
        ],
        out_specs=(
            pl.BlockSpec((1, Cb, H, W), lambda l, c: (l, c, 0, 0)),
            pl.BlockSpec((1, Cb, H, W), lambda l, c: (l, c, 0, 0)),
            pl.BlockSpec(memory_space=pltpu.SMEM),
        ),
        out_shape=(
            jax.ShapeDtypeStruct((L, C, H, W), jnp.float32),
            jax.ShapeDtypeStruct((L, C, H, W), jnp.float32),
            jax.ShapeDtypeStruct((1, 1), jnp.float32),
        ),
        scratch_shapes=[
            pltpu.VMEM((4, H, W), jnp.float32),
            pltpu.VMEM((H, W), jnp.float32),
        ],
    )(confidence_maps, targets_label, jnp.asarray(_BNP, jnp.bfloat16), x)
    return xc, xg, rate[0, 0]
